# baseline (speedup 1.0000x reference)
import functools

import jax
import jax.numpy as jnp
from jax import lax
from jax.experimental import pallas as pl
from jax.experimental.pallas import tpu as pltpu
from jax.experimental.pallas import tpu_sc as plsc

B = 4
H = 76
W = 76
A = 9
M = 50
C = 80
PAD = 608.0
HW = H * W
N = HW * A
IGNORE_IOU = 0.7

MP = 56
MSC = 64
LB = 4096
NBL = -(-N // LB)

_LN2 = 0.6931471805599453


def _log16(u):
    v = u
    acc = jnp.zeros((16,), jnp.float32)
    for _ in range(5):
        c = v > 1.41421356
        v = jnp.where(c, v * 0.5, v)
        acc = jnp.where(c, acc + _LN2, acc)
    for _ in range(9):
        c = v < 0.70710678
        v = jnp.where(c, v * 2.0, v)
        acc = jnp.where(c, acc - _LN2, acc)
    z = (v - 1.0) / (v + 1.0)
    z2 = z * z
    return acc + 2.0 * z * (1.0 + z2 * (1.0 / 3.0 + z2 * (0.2 + z2 / 7.0)))


def _sc_body(gt_hbm, anc_hbm, rows_hbm, tx_hbm, ty_hbm, sw_hbm, sh_hbm,
             wg_hbm, gt_v, anc_v, rows_v, tx_v, ty_v, sw_v, sh_v, wg_v):
    wid = lax.axis_index("s") * 2 + lax.axis_index("c")

    @pl.when(wid < B)
    def _():
        b = wid
        pltpu.sync_copy(gt_hbm.at[pl.ds(b, 1)], gt_v)
        pltpu.sync_copy(anc_hbm, anc_v)
        for k in range(MSC // 16):
            s = pl.ds(k * 16, 16)
            x0 = gt_v[0, 0, s]
            y0 = gt_v[0, 1, s]
            x1 = gt_v[0, 2, s]
            y1 = gt_v[0, 3, s]
            gtx = (x0 + x1) * 0.5
            gty = (y0 + y1) * 0.5
            gtw = x1 - x0
            gth = y1 - y0
            ag = gtw * gth
            best_iou = jnp.full((16,), -1.0, jnp.float32)
            best_a = jnp.zeros((16,), jnp.int32)
            best_aw = jnp.full((16,), 1.0, jnp.float32)
            best_ah = jnp.full((16,), 1.0, jnp.float32)
            for a in range(A):
                aw = anc_v[a, :]
                ah = anc_v[A + a, :]
                mi = jnp.minimum(aw, gtw) * jnp.minimum(ah, gth)
                iou = mi / (aw * ah + ag - mi + 1e-12)
                better = iou > best_iou
                best_iou = jnp.where(better, iou, best_iou)
                best_a = jnp.where(better, a, best_a)
                best_aw = jnp.where(better, aw, best_aw)
                best_ah = jnp.where(better, ah, best_ah)
            valid = (x0 >= 0.0) & (y0 >= 0.0) & (x1 >= 0.0) & (y1 >= 0.0)
            lxf = gtx / PAD * W
            lyf = gty / PAD * H
            loc_x = jnp.clip(lxf.astype(jnp.int32), 0, W - 1)
            loc_y = jnp.clip(lyf.astype(jnp.int32), 0, H - 1)
            index = jnp.where(valid, loc_y * W + loc_x, HW)
            rows_v[s] = index * A + best_a
            tx_v[s] = lxf - loc_x.astype(jnp.float32)
            ty_v[s] = lyf - loc_y.astype(jnp.float32)
            sw_v[s] = _log16(jnp.maximum(gtw, 1.0) / best_aw)
            sh_v[s] = _log16(jnp.maximum(gth, 1.0) / best_ah)
            wg_v[s] = 2.0 - gtw * gth / PAD / PAD
        pltpu.sync_copy(rows_v, rows_hbm.at[b])
        pltpu.sync_copy(tx_v, tx_hbm.at[b])
        pltpu.sync_copy(ty_v, ty_hbm.at[b])
        pltpu.sync_copy(sw_v, sw_hbm.at[b])
        pltpu.sync_copy(sh_v, sh_hbm.at[b])
        pltpu.sync_copy(wg_v, wg_hbm.at[b])


_sc_match = functools.partial(
    pl.kernel,
    out_type=[jax.ShapeDtypeStruct((B, MSC), jnp.int32)] +
             [jax.ShapeDtypeStruct((B, MSC), jnp.float32)] * 5,
    mesh=plsc.VectorSubcoreMesh(core_axis_name="c", subcore_axis_name="s"),
    scratch_types=[
        pltpu.VMEM((1, 4, MSC), jnp.float32),
        pltpu.VMEM((2 * A, 16), jnp.float32),
        pltpu.VMEM((MSC,), jnp.int32),
        pltpu.VMEM((MSC,), jnp.float32),
        pltpu.VMEM((MSC,), jnp.float32),
        pltpu.VMEM((MSC,), jnp.float32),
        pltpu.VMEM((MSC,), jnp.float32),
        pltpu.VMEM((MSC,), jnp.float32),
    ],
)(_sc_body)


def _tc_body(boxt_ref, gt_ref, lab_ref, row_ref, tx_ref, ty_ref, sw_ref,
             sh_ref, wg_ref, obj_ref, cen_ref, sca_ref, wei_ref, cls_ref):
    i = pl.program_id(1)

    boxt = boxt_ref[0]
    gt = gt_ref[0]
    lab = lab_ref[0]
    row = row_ref[0]
    tx = tx_ref[0]
    ty = ty_ref[0]
    sw = sw_ref[0]
    sh = sh_ref[0]
    wgt = wg_ref[0]

    c_iota = jax.lax.broadcasted_iota(jnp.int32, (MP, C + 1), 1)
    lmat = ((lab - 1) == c_iota).astype(jnp.float32) + \
        (c_iota == C).astype(jnp.float32)

    ridx = i * LB + jax.lax.broadcasted_iota(jnp.int32, (1, LB), 1)
    eq = row == ridx
    eqf = eq.astype(jnp.float32)
    m_iota = jax.lax.broadcasted_iota(jnp.int32, (MP, 1), 0)
    win = jnp.max(jnp.where(eq, jnp.broadcast_to(m_iota, (MP, LB)), -1),
                  axis=0, keepdims=True)
    hit = win >= 0
    ohwf = (m_iota == win).astype(jnp.float32)
    txv = jnp.sum(ohwf * tx, axis=0, keepdims=True)
    tyv = jnp.sum(ohwf * ty, axis=0, keepdims=True)
    swv = jnp.sum(ohwf * sw, axis=0, keepdims=True)
    shv = jnp.sum(ohwf * sh, axis=0, keepdims=True)
    wgv = jnp.sum(ohwf * wgt, axis=0, keepdims=True)

    dims = (((0,), (0,)), ((), ()))
    counts = jax.lax.dot_general(eqf, lmat, dims,
                                 preferred_element_type=jnp.float32)
    anyeq = counts[:, C:C + 1] > 0.5
    cls = jnp.where(anyeq, jnp.minimum(counts[:, :C], 1.0), -1.0)

    gx0 = gt[:, 0:1]
    gy0 = gt[:, 1:2]
    gx1 = gt[:, 2:3]
    gy1 = gt[:, 3:4]
    px0 = boxt[0:1, :]
    py0 = boxt[1:2, :]
    px1 = boxt[2:3, :]
    py1 = boxt[3:4, :]
    itlx = jnp.maximum(px0, gx0)
    itly = jnp.maximum(py0, gy0)
    ibrx = jnp.minimum(px1, gx1)
    ibry = jnp.minimum(py1, gy1)
    iiw = jnp.maximum(ibrx - itlx, 0.0)
    iih = jnp.maximum(ibry - itly, 0.0)
    pinter = iiw * iih
    parea = (px1 - px0) * (py1 - py0)
    garea = (gx1 - gx0) * (gy1 - gy0)
    piou = pinter / (parea + garea - pinter + 1e-12)
    pmax = jnp.max(piou, axis=0, keepdims=True)
    dyn = jnp.where(pmax > IGNORE_IOU, -1.0, 0.0)

    obj_ref[0] = jnp.where(hit, 1.0, dyn)
    cen_ref[0] = jnp.where(hit, jnp.concatenate([txv, tyv], axis=0), 0.0)
    sca_ref[0] = jnp.where(hit, jnp.concatenate([swv, shv], axis=0), 0.0)
    wei_ref[0] = jnp.where(hit, jnp.concatenate([wgv, wgv], axis=0), 0.0)
    cls_ref[0] = cls


def kernel(box_preds, gt_boxes, anchors, gt_labels):
    gt_t = jnp.transpose(gt_boxes, (0, 2, 1))
    gt_sc = jnp.pad(gt_t, ((0, 0), (0, 0), (0, MSC - M)),
                    constant_values=-1.0)
    anc_b = jnp.tile(jnp.transpose(anchors, (1, 0)).reshape(2 * A, 1),
                     (1, 16))
    rows, txa, tya, swa, sha, wga = _sc_match(gt_sc, anc_b)

    col = lambda x: x[:, :MP].reshape(B, MP, 1)
    lab = jnp.pad(gt_labels, ((0, 0), (0, MP - M))).reshape(B, MP, 1)
    box_t = jnp.transpose(box_preds, (0, 2, 1))

    grid = (B, NBL)
    out = pl.pallas_call(
        _tc_body,
        grid=grid,
        in_specs=[
            pl.BlockSpec((1, 4, LB), lambda b, i: (b, 0, i)),
            pl.BlockSpec((1, M, 4), lambda b, i: (b, 0, 0)),
            pl.BlockSpec((1, MP, 1), lambda b, i: (b, 0, 0)),
            pl.BlockSpec((1, MP, 1), lambda b, i: (b, 0, 0)),
            pl.BlockSpec((1, MP, 1), lambda b, i: (b, 0, 0)),
            pl.BlockSpec((1, MP, 1), lambda b, i: (b, 0, 0)),
            pl.BlockSpec((1, MP, 1), lambda b, i: (b, 0, 0)),
            pl.BlockSpec((1, MP, 1), lambda b, i: (b, 0, 0)),
            pl.BlockSpec((1, MP, 1), lambda b, i: (b, 0, 0)),
        ],
        out_specs=[
            pl.BlockSpec((1, 1, LB), lambda b, i: (b, 0, i)),
            pl.BlockSpec((1, 2, LB), lambda b, i: (b, 0, i)),
            pl.BlockSpec((1, 2, LB), lambda b, i: (b, 0, i)),
            pl.BlockSpec((1, 2, LB), lambda b, i: (b, 0, i)),
            pl.BlockSpec((1, LB, C), lambda b, i: (b, i, 0)),
        ],
        out_shape=[
            jax.ShapeDtypeStruct((B, 1, N), jnp.float32),
            jax.ShapeDtypeStruct((B, 2, N), jnp.float32),
            jax.ShapeDtypeStruct((B, 2, N), jnp.float32),
            jax.ShapeDtypeStruct((B, 2, N), jnp.float32),
            jax.ShapeDtypeStruct((B, N, C), jnp.float32),
        ],
        compiler_params=pltpu.CompilerParams(
            dimension_semantics=("parallel", "parallel"),
        ),
    )(box_t, gt_boxes, lab, col(rows), col(txa), col(tya), col(swa),
      col(sha), col(wga))
    obj, cen, sca, wei, cls = out
    tr = lambda x: jnp.transpose(x, (0, 2, 1))
    return (obj.reshape(B, N, 1), tr(cen), tr(sca), tr(wei), cls)


# speedup vs baseline: 1.0109x; 1.0109x over previous
import functools

import jax
import jax.numpy as jnp
from jax import lax
from jax.experimental import pallas as pl
from jax.experimental.pallas import tpu as pltpu
from jax.experimental.pallas import tpu_sc as plsc

B = 4
H = 76
W = 76
A = 9
M = 50
C = 80
PAD = 608.0
HW = H * W
N = HW * A
IGNORE_IOU = 0.7

MP = 56
MSC = 64
LB = 4096
NBL = -(-N // LB)

_LN2 = 0.6931471805599453


def _log16(u):
    v = u
    acc = jnp.zeros((16,), jnp.float32)
    for _ in range(5):
        c = v > 1.41421356
        v = jnp.where(c, v * 0.5, v)
        acc = jnp.where(c, acc + _LN2, acc)
    for _ in range(9):
        c = v < 0.70710678
        v = jnp.where(c, v * 2.0, v)
        acc = jnp.where(c, acc - _LN2, acc)
    z = (v - 1.0) / (v + 1.0)
    z2 = z * z
    return acc + 2.0 * z * (1.0 + z2 * (1.0 / 3.0 + z2 * (0.2 + z2 / 7.0)))


def _sc_body(gt_hbm, anc_hbm, rows_hbm, tx_hbm, ty_hbm, sw_hbm, sh_hbm,
             wg_hbm, gt_v, anc_v, rows_v, tx_v, ty_v, sw_v, sh_v, wg_v,
             txo_v, tyo_v, swo_v, sho_v, wgo_v):
    wid = lax.axis_index("s") * 2 + lax.axis_index("c")

    @pl.when(wid < B)
    def _():
        b = wid
        pltpu.sync_copy(gt_hbm.at[pl.ds(b, 1)], gt_v)
        pltpu.sync_copy(anc_hbm, anc_v)
        for k in range(MSC // 16):
            s = pl.ds(k * 16, 16)
            x0 = gt_v[0, 0, s]
            y0 = gt_v[0, 1, s]
            x1 = gt_v[0, 2, s]
            y1 = gt_v[0, 3, s]
            gtx = (x0 + x1) * 0.5
            gty = (y0 + y1) * 0.5
            gtw = x1 - x0
            gth = y1 - y0
            ag = gtw * gth
            best_iou = jnp.full((16,), -1.0, jnp.float32)
            best_a = jnp.zeros((16,), jnp.int32)
            best_aw = jnp.full((16,), 1.0, jnp.float32)
            best_ah = jnp.full((16,), 1.0, jnp.float32)
            for a in range(A):
                aw = anc_v[a, :]
                ah = anc_v[A + a, :]
                mi = jnp.minimum(aw, gtw) * jnp.minimum(ah, gth)
                iou = mi / (aw * ah + ag - mi + 1e-12)
                better = iou > best_iou
                best_iou = jnp.where(better, iou, best_iou)
                best_a = jnp.where(better, a, best_a)
                best_aw = jnp.where(better, aw, best_aw)
                best_ah = jnp.where(better, ah, best_ah)
            valid = (x0 >= 0.0) & (y0 >= 0.0) & (x1 >= 0.0) & (y1 >= 0.0)
            lxf = gtx / PAD * W
            lyf = gty / PAD * H
            loc_x = jnp.clip(lxf.astype(jnp.int32), 0, W - 1)
            loc_y = jnp.clip(lyf.astype(jnp.int32), 0, H - 1)
            index = jnp.where(valid, loc_y * W + loc_x, HW)
            rows_v[s] = index * A + best_a
            tx_v[s] = lxf - loc_x.astype(jnp.float32)
            ty_v[s] = lyf - loc_y.astype(jnp.float32)
            sw_v[s] = _log16(jnp.maximum(gtw, 1.0) / best_aw)
            sh_v[s] = _log16(jnp.maximum(gth, 1.0) / best_ah)
            wg_v[s] = 2.0 - gtw * gth / PAD / PAD
        nch = MSC // 16
        _dn = lax.GatherDimensionNumbers(
            offset_dims=(), collapsed_slice_dims=(0,), start_index_map=(0,))
        tk = lambda v, idx: lax.gather(
            v, idx[:, None], _dn, slice_sizes=(1,),
            mode=lax.GatherScatterMode.PROMISE_IN_BOUNDS)
        rch = [rows_v[pl.ds(d * 16, 16)] for d in range(nch)]
        for k in range(nch):
            s = pl.ds(k * 16, 16)
            rowc = rch[k]
            win = lax.iota(jnp.int32, 16) + k * 16
            cnt = jnp.zeros((16,), jnp.float32)
            for d in range(nch):
                for off in range(16):
                    perm = (lax.iota(jnp.int32, 16) + off) & 15
                    rj = tk(rch[d], perm)
                    jvec = perm + d * 16
                    same = rj == rowc
                    win = jnp.where(same & (jvec > win), jvec, win)
                    cnt = cnt + jnp.where(same, 1.0, 0.0)
            wl = win & 15
            wc = win >> 4
            for src_v, dst_v in ((tx_v, txo_v), (ty_v, tyo_v), (sw_v, swo_v),
                                 (sh_v, sho_v), (wg_v, wgo_v)):
                val = jnp.zeros((16,), jnp.float32)
                for d in range(nch):
                    vd = tk(src_v[pl.ds(d * 16, 16)], wl)
                    val = jnp.where(wc == d, vd, val)
                dst_v[s] = val / cnt
        pltpu.sync_copy(rows_v, rows_hbm.at[b])
        pltpu.sync_copy(txo_v, tx_hbm.at[b])
        pltpu.sync_copy(tyo_v, ty_hbm.at[b])
        pltpu.sync_copy(swo_v, sw_hbm.at[b])
        pltpu.sync_copy(sho_v, sh_hbm.at[b])
        pltpu.sync_copy(wgo_v, wg_hbm.at[b])


_sc_match = functools.partial(
    pl.kernel,
    out_type=[jax.ShapeDtypeStruct((B, MSC), jnp.int32)] +
             [jax.ShapeDtypeStruct((B, MSC), jnp.float32)] * 5,
    mesh=plsc.VectorSubcoreMesh(core_axis_name="c", subcore_axis_name="s"),
    scratch_types=[
        pltpu.VMEM((1, 4, MSC), jnp.float32),
        pltpu.VMEM((2 * A, 16), jnp.float32),
        pltpu.VMEM((MSC,), jnp.int32),
        pltpu.VMEM((MSC,), jnp.float32),
        pltpu.VMEM((MSC,), jnp.float32),
        pltpu.VMEM((MSC,), jnp.float32),
        pltpu.VMEM((MSC,), jnp.float32),
        pltpu.VMEM((MSC,), jnp.float32),
        pltpu.VMEM((MSC,), jnp.float32),
        pltpu.VMEM((MSC,), jnp.float32),
        pltpu.VMEM((MSC,), jnp.float32),
        pltpu.VMEM((MSC,), jnp.float32),
        pltpu.VMEM((MSC,), jnp.float32),
    ],
)(_sc_body)


def _tc_body(boxt_ref, gt_ref, lab_ref, row_ref, tx_ref, ty_ref, sw_ref,
             sh_ref, wg_ref, obj_ref, cen_ref, sca_ref, wei_ref, cls_ref):
    i = pl.program_id(1)

    boxt = boxt_ref[0]
    gt = gt_ref[0]
    lab = lab_ref[0]
    row = row_ref[0]
    tx = tx_ref[0]
    ty = ty_ref[0]
    sw = sw_ref[0]
    sh = sh_ref[0]
    wgt = wg_ref[0]

    c_iota = jax.lax.broadcasted_iota(jnp.int32, (MP, C + 1), 1)
    lmat = ((lab - 1) == c_iota).astype(jnp.float32) + \
        (c_iota == C).astype(jnp.float32)

    ridx = i * LB + jax.lax.broadcasted_iota(jnp.int32, (1, LB), 1)
    eq = row == ridx
    eqf = eq.astype(jnp.float32)
    hit = jnp.max(eqf, axis=0, keepdims=True) > 0.5
    txv = jnp.sum(eqf * tx, axis=0, keepdims=True)
    tyv = jnp.sum(eqf * ty, axis=0, keepdims=True)
    swv = jnp.sum(eqf * sw, axis=0, keepdims=True)
    shv = jnp.sum(eqf * sh, axis=0, keepdims=True)
    wgv = jnp.sum(eqf * wgt, axis=0, keepdims=True)

    dims = (((0,), (0,)), ((), ()))
    counts = jax.lax.dot_general(eqf, lmat, dims,
                                 preferred_element_type=jnp.float32)
    anyeq = counts[:, C:C + 1] > 0.5
    cls = jnp.where(anyeq, jnp.minimum(counts[:, :C], 1.0), -1.0)

    gx0 = gt[:, 0:1]
    gy0 = gt[:, 1:2]
    gx1 = gt[:, 2:3]
    gy1 = gt[:, 3:4]
    px0 = boxt[0:1, :]
    py0 = boxt[1:2, :]
    px1 = boxt[2:3, :]
    py1 = boxt[3:4, :]
    itlx = jnp.maximum(px0, gx0)
    itly = jnp.maximum(py0, gy0)
    ibrx = jnp.minimum(px1, gx1)
    ibry = jnp.minimum(py1, gy1)
    iiw = jnp.maximum(ibrx - itlx, 0.0)
    iih = jnp.maximum(ibry - itly, 0.0)
    pinter = iiw * iih
    parea = (px1 - px0) * (py1 - py0)
    garea = (gx1 - gx0) * (gy1 - gy0)
    piou = pinter / (parea + garea - pinter + 1e-12)
    pmax = jnp.max(piou, axis=0, keepdims=True)
    dyn = jnp.where(pmax > IGNORE_IOU, -1.0, 0.0)

    obj_ref[0] = jnp.where(hit, 1.0, dyn)
    cen_ref[0] = jnp.where(hit, jnp.concatenate([txv, tyv], axis=0), 0.0)
    sca_ref[0] = jnp.where(hit, jnp.concatenate([swv, shv], axis=0), 0.0)
    wei_ref[0] = jnp.where(hit, jnp.concatenate([wgv, wgv], axis=0), 0.0)
    cls_ref[0] = cls


def kernel(box_preds, gt_boxes, anchors, gt_labels):
    gt_t = jnp.transpose(gt_boxes, (0, 2, 1))
    gt_sc = jnp.pad(gt_t, ((0, 0), (0, 0), (0, MSC - M)),
                    constant_values=-1.0)
    anc_b = jnp.tile(jnp.transpose(anchors, (1, 0)).reshape(2 * A, 1),
                     (1, 16))
    rows, txa, tya, swa, sha, wga = _sc_match(gt_sc, anc_b)

    col = lambda x: x[:, :MP].reshape(B, MP, 1)
    lab = jnp.pad(gt_labels, ((0, 0), (0, MP - M))).reshape(B, MP, 1)
    box_t = jnp.transpose(box_preds, (0, 2, 1))

    grid = (B, NBL)
    out = pl.pallas_call(
        _tc_body,
        grid=grid,
        in_specs=[
            pl.BlockSpec((1, 4, LB), lambda b, i: (b, 0, i)),
            pl.BlockSpec((1, M, 4), lambda b, i: (b, 0, 0)),
            pl.BlockSpec((1, MP, 1), lambda b, i: (b, 0, 0)),
            pl.BlockSpec((1, MP, 1), lambda b, i: (b, 0, 0)),
            pl.BlockSpec((1, MP, 1), lambda b, i: (b, 0, 0)),
            pl.BlockSpec((1, MP, 1), lambda b, i: (b, 0, 0)),
            pl.BlockSpec((1, MP, 1), lambda b, i: (b, 0, 0)),
            pl.BlockSpec((1, MP, 1), lambda b, i: (b, 0, 0)),
            pl.BlockSpec((1, MP, 1), lambda b, i: (b, 0, 0)),
        ],
        out_specs=[
            pl.BlockSpec((1, 1, LB), lambda b, i: (b, 0, i)),
            pl.BlockSpec((1, 2, LB), lambda b, i: (b, 0, i)),
            pl.BlockSpec((1, 2, LB), lambda b, i: (b, 0, i)),
            pl.BlockSpec((1, 2, LB), lambda b, i: (b, 0, i)),
            pl.BlockSpec((1, LB, C), lambda b, i: (b, i, 0)),
        ],
        out_shape=[
            jax.ShapeDtypeStruct((B, 1, N), jnp.float32),
            jax.ShapeDtypeStruct((B, 2, N), jnp.float32),
            jax.ShapeDtypeStruct((B, 2, N), jnp.float32),
            jax.ShapeDtypeStruct((B, 2, N), jnp.float32),
            jax.ShapeDtypeStruct((B, N, C), jnp.float32),
        ],
        compiler_params=pltpu.CompilerParams(
            dimension_semantics=("parallel", "parallel"),
        ),
    )(box_t, gt_boxes, lab, col(rows), col(txa), col(tya), col(swa),
      col(sha), col(wga))
    obj, cen, sca, wei, cls = out
    tr = lambda x: jnp.transpose(x, (0, 2, 1))
    return (obj.reshape(B, N, 1), tr(cen), tr(sca), tr(wei), cls)
